# baseline (device time: 9656 ns/iter reference)
import jax
import jax.numpy as jnp
from jax import lax
from jax.experimental import pallas as pl
from jax.experimental.pallas import tpu as pltpu

N_DEV = 4
CHUNK = 256


def kernel(x):
    m_per, n = x.shape
    n_chunks = m_per // CHUNK

    def body(x_hbm, out_ref, acc_ref, vbuf, comm_ref,
             copy_sems, send_sems, recv_sems):
        my_pos = lax.axis_index("i")
        peers = [(my_pos + k) % N_DEV for k in (1, 2, 3)]

        barrier_sem = pltpu.get_barrier_semaphore()
        for p in peers:
            pl.semaphore_signal(
                barrier_sem, inc=1,
                device_id=(p,), device_id_type=pl.DeviceIdType.MESH,
            )

        def copy_in(i):
            return pltpu.make_async_copy(
                x_hbm.at[pl.ds(i * CHUNK, CHUNK), :],
                vbuf.at[i % 2],
                copy_sems.at[i % 2],
            )

        copy_in(0).start()
        copy_in(1).start()
        acc = None
        for i in range(n_chunks):
            copy_in(i).wait()
            part = jnp.max(vbuf[i % 2], axis=0, keepdims=True)
            acc = part if acc is None else jnp.maximum(acc, part)
            if i + 2 < n_chunks:
                copy_in(i + 2).start()
        acc_ref[:, :] = acc

        pl.semaphore_wait(barrier_sem, 3)

        rdmas = []
        for k in (1, 2, 3):
            r = pltpu.make_async_remote_copy(
                src_ref=acc_ref,
                dst_ref=comm_ref.at[3 - k],
                send_sem=send_sems.at[k - 1],
                recv_sem=recv_sems.at[3 - k],
                device_id=(peers[k - 1],),
                device_id_type=pl.DeviceIdType.MESH,
            )
            r.start()
            rdmas.append(r)
        for r in rdmas:
            r.wait()

        out_ref[:, :] = jnp.maximum(
            jnp.maximum(acc, comm_ref[0, :, :]),
            jnp.maximum(comm_ref[1, :, :], comm_ref[2, :, :]),
        )

    return pl.pallas_call(
        body,
        out_shape=jax.ShapeDtypeStruct((1, n), jnp.float32),
        in_specs=[pl.BlockSpec(memory_space=pltpu.MemorySpace.HBM)],
        out_specs=pl.BlockSpec(memory_space=pltpu.VMEM),
        scratch_shapes=[
            pltpu.VMEM((1, n), jnp.float32),
            pltpu.VMEM((2, CHUNK, n), jnp.float32),
            pltpu.VMEM((3, 1, n), jnp.float32),
            pltpu.SemaphoreType.DMA((2,)),
            pltpu.SemaphoreType.DMA((3,)),
            pltpu.SemaphoreType.DMA((3,)),
        ],
        compiler_params=pltpu.CompilerParams(collective_id=0),
    )(x)
